# Initial kernel scaffold; baseline (speedup 1.0000x reference)
import functools

import jax
import jax.numpy as jnp
import numpy as np
from jax import lax
from jax.experimental import pallas as pl
from jax.experimental.pallas import tpu as pltpu
from jax.experimental.pallas import tpu_sc as plsc

NC = 2
NS = 16
L = 16
NW = NC * NS

B = 4
N = 4096
CHUNKS = N // L
PTS_PER_W = (B * N) // NW
W_PER_B = N // PTS_PER_W

R2 = np.float32(0.1 ** 2)
H2 = np.float32(0.12 ** 2)
RADIUS = np.float32(0.1)
NSAMPLE = 9
INF = np.float32(np.inf)


def _sqrt16(x):
    i = plsc.bitcast(x, jnp.int32)
    i = jnp.int32(0x1FBD1DF5) + lax.shift_right_arithmetic(i, 1)
    y = plsc.bitcast(i, jnp.float32)
    for _ in range(3):
        y = jnp.float32(0.5) * (y + x / y)
    return y


mesh = plsc.VectorSubcoreMesh(core_axis_name="c", subcore_axis_name="s")


@functools.partial(
    pl.kernel,
    out_type=jax.ShapeDtypeStruct((NW, L), jnp.float32),
    mesh=mesh,
    scratch_types=[
        pltpu.VMEM((N,), jnp.float32),
        pltpu.VMEM((N,), jnp.float32),
        pltpu.VMEM((N,), jnp.float32),
        pltpu.VMEM((N,), jnp.float32),
        pltpu.VMEM((N + L,), jnp.float32),
        pltpu.VMEM((L,), jnp.float32),
    ],
)
def _density_sc(coords_hbm, out_hbm, xs, ys, zs, sq, buf, accv):
    w = lax.axis_index("s") * NC + lax.axis_index("c")
    b = w // W_PER_B
    i0 = (w % W_PER_B) * PTS_PER_W

    pltpu.sync_copy(coords_hbm.at[3 * b + 0], xs)
    pltpu.sync_copy(coords_hbm.at[3 * b + 1], ys)
    pltpu.sync_copy(coords_hbm.at[3 * b + 2], zs)

    def sq_body(c, carry):
        sl = pl.ds(c * L, L)
        xv = xs[sl]
        yv = ys[sl]
        zv = zs[sl]
        sq[sl] = (xv * xv + yv * yv) + zv * zv
        return carry

    lax.fori_loop(0, CHUNKS, sq_body, 0)

    lane = lax.iota(jnp.int32, L)

    def point_body(t, acc):
        i = i0 + t
        px = xs[i]
        py = ys[i]
        pz = zs[i]
        si = sq[i]

        def chunk_body(c, cnt):
            sl = pl.ds(c * L, L)
            dot = (px * xs[sl] + py * ys[sl]) + pz * zs[sl]
            d = (si + sq[sl]) - jnp.float32(2.0) * dot
            m = d <= R2
            plsc.store_compressed(buf.at[pl.ds(cnt, L)], d, mask=m)
            return cnt + jnp.sum(m.astype(jnp.int32))

        cnt = lax.fori_loop(0, CHUNKS, chunk_body, jnp.int32(0))

        vals = buf[pl.ds(0, L)]
        first = buf[0]
        kk = jnp.minimum(cnt, NSAMPLE)
        nine = jnp.where(lane < kk, vals, jnp.where(lane < NSAMPLE, first, INF))
        srt = jnp.sort(nine)
        sel = (lane >= 1) & (lane <= 4)
        v = jnp.where(sel, srt, jnp.float32(1.0))
        v = jnp.maximum(v, jnp.float32(1e-12))
        term = RADIUS - _sqrt16(v) * jnp.exp(-v / H2)
        return acc + jnp.where(sel, term, jnp.float32(0.0))

    acc = lax.fori_loop(0, PTS_PER_W, point_body, jnp.zeros((L,), jnp.float32))
    accv[...] = acc
    pltpu.sync_copy(accv, out_hbm.at[w])


def kernel(pred):
    coords = jnp.transpose(pred, (0, 2, 1)).reshape(B * 3, N)
    partials = _density_sc(coords)
    return jnp.sum(partials) / np.float32(B * N * 4)


# speedup vs baseline: 5.8727x; 5.8727x over previous
import functools

import jax
import jax.numpy as jnp
import numpy as np
from jax import lax
from jax.experimental import pallas as pl
from jax.experimental.pallas import tpu as pltpu
from jax.experimental.pallas import tpu_sc as plsc

NC = 2
NS = 16
L = 16
NW = NC * NS

B = 4
N = 4096
CHUNKS = N // L
PTS_PER_W = (B * N) // NW
W_PER_B = N // PTS_PER_W

R2 = np.float32(0.1 ** 2)
H2 = np.float32(0.12 ** 2)
RADIUS = np.float32(0.1)
NSAMPLE = 9
INF = np.float32(np.inf)


def _rne_bf16(x):
    u = plsc.bitcast(x, jnp.int32)
    u = u + jnp.int32(0x7FFF) + (lax.shift_right_logical(u, 16) & jnp.int32(1))
    u = u & jnp.int32(-65536)
    return plsc.bitcast(u, jnp.float32)


def _sqrt16(x):
    i = plsc.bitcast(x, jnp.int32)
    i = jnp.int32(0x1FBD1DF5) + lax.shift_right_arithmetic(i, 1)
    y = plsc.bitcast(i, jnp.float32)
    for _ in range(3):
        y = jnp.float32(0.5) * (y + x / y)
    return y


mesh = plsc.VectorSubcoreMesh(core_axis_name="c", subcore_axis_name="s")


@functools.partial(
    pl.kernel,
    out_type=jax.ShapeDtypeStruct((NW, L), jnp.float32),
    mesh=mesh,
    compiler_params=pltpu.CompilerParams(needs_layout_passes=False),
    scratch_types=[
        pltpu.VMEM((N + L,), jnp.float32),
        pltpu.VMEM((N + L,), jnp.float32),
        pltpu.VMEM((N + L,), jnp.float32),
        pltpu.VMEM((N + L,), jnp.float32),
        pltpu.VMEM((N + L,), jnp.float32),
        pltpu.VMEM((N + L,), jnp.float32),
        pltpu.VMEM((N + L,), jnp.float32),
        pltpu.VMEM((N + L,), jnp.float32),
        pltpu.VMEM((L,), jnp.float32),
    ],
)
def _density_sc(coords_hbm, out_hbm, xs, ys, zs, xb, yb, zb, sq, buf, accv):
    w = lax.axis_index("s") * NC + lax.axis_index("c")
    b = w // W_PER_B
    i0 = (w % W_PER_B) * PTS_PER_W

    pltpu.sync_copy(coords_hbm.at[3 * b + 0], xs.at[pl.ds(0, N)])
    pltpu.sync_copy(coords_hbm.at[3 * b + 1], ys.at[pl.ds(0, N)])
    pltpu.sync_copy(coords_hbm.at[3 * b + 2], zs.at[pl.ds(0, N)])

    def sq_body(c, carry):
        sl = pl.ds(c * L, L)
        xv = xs[sl]
        yv = ys[sl]
        zv = zs[sl]
        sq[sl] = (xv * xv + yv * yv) + zv * zv
        xb[sl] = _rne_bf16(xv)
        yb[sl] = _rne_bf16(yv)
        zb[sl] = _rne_bf16(zv)
        return carry

    lax.fori_loop(0, CHUNKS, sq_body, 0)

    lane = lax.iota(jnp.int32, L)

    def point_body(t, acc):
        i = i0 + t
        px = xs[pl.ds(i, L)][0]
        py = ys[pl.ds(i, L)][0]
        pz = zs[pl.ds(i, L)][0]
        pxr = xb[pl.ds(i, L)][0]
        pyr = yb[pl.ds(i, L)][0]
        pzr = zb[pl.ds(i, L)][0]
        si = sq[pl.ds(i, L)][0]

        def chunk_body(c, cnt):
            sl = pl.ds(c * L, L)
            dot = (pxr * xb[sl] + pyr * yb[sl]) + pzr * zb[sl]
            d = (si + sq[sl]) - jnp.float32(2.0) * dot
            m = d <= R2
            dx = xs[sl] - px
            dy = ys[sl] - py
            dz = zs[sl] - pz
            v = (dx * dx + dy * dy) + dz * dz
            csum = plsc.cumsum(m.astype(jnp.int32))
            plsc.store_scatter(buf, [cnt + csum - 1], v, mask=m)
            return cnt + csum[L - 1]

        cnt = lax.fori_loop(0, CHUNKS, chunk_body, jnp.int32(0))

        vals = buf[pl.ds(0, L)]
        lsl = pl.ds(N - L, L)
        ex = xs[lsl] - px
        ey = ys[lsl] - py
        ez = zs[lsl] - pz
        dlast = ((ex * ex + ey * ey) + ez * ez)[L - 1]
        first = jnp.where(cnt == 0, dlast, vals[0])
        kk = jnp.minimum(cnt, NSAMPLE)
        nine = jnp.where(lane < kk, vals, jnp.where(lane < NSAMPLE, first, INF))
        srt = jnp.sort(nine)
        sel = (lane >= 1) & (lane <= 4)
        v = jnp.where(sel, srt, jnp.float32(1.0))
        v = jnp.maximum(v, jnp.float32(1e-12))
        term = RADIUS - _sqrt16(v) * jnp.exp(-v / H2)
        return acc + jnp.where(sel, term, jnp.float32(0.0))

    acc = lax.fori_loop(0, PTS_PER_W, point_body, jnp.zeros((L,), jnp.float32))
    accv[...] = acc
    pltpu.sync_copy(accv, out_hbm.at[w])


def kernel(pred):
    coords = jnp.transpose(pred, (0, 2, 1)).reshape(B * 3, N)
    partials = _density_sc(coords)
    return jnp.sum(partials) / np.float32(B * N * 4)


# speedup vs baseline: 20.5218x; 3.4944x over previous
import functools

import jax
import jax.numpy as jnp
import numpy as np
from jax import lax
from jax.experimental import pallas as pl
from jax.experimental.pallas import tpu as pltpu
from jax.experimental.pallas import tpu_sc as plsc

NC = 2
NS = 16
L = 16
NW = NC * NS

B = 4
N = 4096
CHUNKS = N // L
PTS_PER_W = (B * N) // NW
W_PER_B = N // PTS_PER_W
G = 4
GROUPS = PTS_PER_W // G
BUFW = N + L

R2 = np.float32(0.1 ** 2)
H2 = np.float32(0.12 ** 2)
RADIUS = np.float32(0.1)
NSAMPLE = 9
INF = np.float32(np.inf)


def _rne_bf16(x):
    u = plsc.bitcast(x, jnp.int32)
    u = u + jnp.int32(0x7FFF) + (lax.shift_right_logical(u, 16) & jnp.int32(1))
    u = u & jnp.int32(-65536)
    return plsc.bitcast(u, jnp.float32)


def _sqrt16(x):
    i = plsc.bitcast(x, jnp.int32)
    i = jnp.int32(0x1FBD1DF5) + lax.shift_right_arithmetic(i, 1)
    y = plsc.bitcast(i, jnp.float32)
    for _ in range(3):
        y = jnp.float32(0.5) * (y + x / y)
    return y


mesh = plsc.VectorSubcoreMesh(core_axis_name="c", subcore_axis_name="s")


@functools.partial(
    pl.kernel,
    out_type=jax.ShapeDtypeStruct((NW, L), jnp.float32),
    mesh=mesh,
    compiler_params=pltpu.CompilerParams(needs_layout_passes=False),
    scratch_types=[
        pltpu.VMEM((N + L,), jnp.float32),
        pltpu.VMEM((N + L,), jnp.float32),
        pltpu.VMEM((N + L,), jnp.float32),
        pltpu.VMEM((N + L,), jnp.float32),
        pltpu.VMEM((N + L,), jnp.float32),
        pltpu.VMEM((N + L,), jnp.float32),
        pltpu.VMEM((N + L,), jnp.float32),
        pltpu.VMEM((G * BUFW,), jnp.int32),
        pltpu.VMEM((L,), jnp.float32),
    ],
)
def _density_sc(coords_hbm, out_hbm, xs, ys, zs, xb, yb, zb, sq, ibuf, accv):
    w = lax.axis_index("s") * NC + lax.axis_index("c")
    b = w // W_PER_B
    i0 = (w % W_PER_B) * PTS_PER_W

    pltpu.sync_copy(coords_hbm.at[3 * b + 0], xs.at[pl.ds(0, N)])
    pltpu.sync_copy(coords_hbm.at[3 * b + 1], ys.at[pl.ds(0, N)])
    pltpu.sync_copy(coords_hbm.at[3 * b + 2], zs.at[pl.ds(0, N)])

    def sq_body(c, carry):
        sl = pl.ds(c * L, L)
        xv = xs[sl]
        yv = ys[sl]
        zv = zs[sl]
        sq[sl] = (xv * xv + yv * yv) + zv * zv
        xb[sl] = _rne_bf16(xv)
        yb[sl] = _rne_bf16(yv)
        zb[sl] = _rne_bf16(zv)
        return carry

    lax.fori_loop(0, CHUNKS, sq_body, 0)

    lane = lax.iota(jnp.int32, L)

    def group_body(t, acc):
        ib = i0 + t * G
        ps = []
        for g in range(G):
            sl = pl.ds(ib + g, L)
            ps.append((jnp.float32(2.0) * xb[sl][0],
                       jnp.float32(2.0) * yb[sl][0],
                       jnp.float32(2.0) * zb[sl][0],
                       sq[sl][0]))

        def chunk_body(c, cnts):
            sl = pl.ds(c * L, L)
            xv = xb[sl]
            yv = yb[sl]
            zv = zb[sl]
            sv = sq[sl]
            gidx = c * L + lane
            out = []
            for g in range(G):
                px2, py2, pz2, si = ps[g]
                dot2 = (px2 * xv + py2 * yv) + pz2 * zv
                d = (si + sv) - dot2
                m = d <= R2
                csum = plsc.cumsum(m.astype(jnp.int32))
                plsc.store_scatter(ibuf, [(g * BUFW - 1 + cnts[g]) + csum],
                                   gidx, mask=m)
                out.append(cnts[g] + csum[L - 1])
            return tuple(out)

        cnts = lax.fori_loop(0, CHUNKS, chunk_body,
                             (jnp.int32(0),) * G, unroll=2)

        for g in range(G):
            i = ib + g
            cnt = cnts[g]
            psl = pl.ds(i, L)
            px = xs[psl][0]
            py = ys[psl][0]
            pz = zs[psl][0]
            idx16 = ibuf[pl.ds(g * BUFW, L)]
            kk = jnp.minimum(cnt, NSAMPLE)
            safe = jnp.where(lane < kk, idx16, 0)
            gx = plsc.load_gather(xs, [safe])
            gy = plsc.load_gather(ys, [safe])
            gz = plsc.load_gather(zs, [safe])
            dx = gx - px
            dy = gy - py
            dz = gz - pz
            vals = (dx * dx + dy * dy) + dz * dz
            lsl = pl.ds(N - L, L)
            ex = xs[lsl] - px
            ey = ys[lsl] - py
            ez = zs[lsl] - pz
            dlast = ((ex * ex + ey * ey) + ez * ez)[L - 1]
            first = jnp.where(cnt == 0, dlast, vals[0])
            nine = jnp.where(lane < kk, vals,
                             jnp.where(lane < NSAMPLE, first, INF))
            srt = jnp.sort(nine)
            sel = (lane >= 1) & (lane <= 4)
            v = jnp.where(sel, srt, jnp.float32(1.0))
            v = jnp.maximum(v, jnp.float32(1e-12))
            term = RADIUS - _sqrt16(v) * jnp.exp(-v / H2)
            acc = acc + jnp.where(sel, term, jnp.float32(0.0))
        return acc

    acc = lax.fori_loop(0, GROUPS, group_body, jnp.zeros((L,), jnp.float32))
    accv[...] = acc
    pltpu.sync_copy(accv, out_hbm.at[w])


def kernel(pred):
    coords = jnp.transpose(pred, (0, 2, 1)).reshape(B * 3, N)
    partials = _density_sc(coords)
    return jnp.sum(partials) / np.float32(B * N * 4)


# speedup vs baseline: 28.4277x; 1.3852x over previous
import functools

import jax
import jax.numpy as jnp
import numpy as np
from jax import lax
from jax.experimental import pallas as pl
from jax.experimental.pallas import tpu as pltpu
from jax.experimental.pallas import tpu_sc as plsc

NC = 2
NS = 16
L = 16
NW = NC * NS

B = 4
N = 4096
CHUNKS = N // L
PTS_PER_W = (B * N) // NW
W_PER_B = N // PTS_PER_W
G = 8
GROUPS = PTS_PER_W // G
BUFW = N + L

R2 = np.float32(0.1 ** 2)
H2 = np.float32(0.12 ** 2)
RADIUS = np.float32(0.1)
NSAMPLE = 9
INF = np.float32(np.inf)


def _rne_bf16(x):
    u = plsc.bitcast(x, jnp.int32)
    u = u + jnp.int32(0x7FFF) + (lax.shift_right_logical(u, 16) & jnp.int32(1))
    u = u & jnp.int32(-65536)
    return plsc.bitcast(u, jnp.float32)


def _sqrt16(x):
    i = plsc.bitcast(x, jnp.int32)
    i = jnp.int32(0x1FBD1DF5) + lax.shift_right_arithmetic(i, 1)
    y = plsc.bitcast(i, jnp.float32)
    for _ in range(3):
        y = jnp.float32(0.5) * (y + x / y)
    return y


mesh = plsc.VectorSubcoreMesh(core_axis_name="c", subcore_axis_name="s")


@functools.partial(
    pl.kernel,
    out_type=jax.ShapeDtypeStruct((NW, L), jnp.float32),
    mesh=mesh,
    compiler_params=pltpu.CompilerParams(needs_layout_passes=False),
    scratch_types=[
        pltpu.VMEM((N + L,), jnp.float32),
        pltpu.VMEM((N + L,), jnp.float32),
        pltpu.VMEM((N + L,), jnp.float32),
        pltpu.VMEM((N + L,), jnp.float32),
        pltpu.VMEM((N + L,), jnp.float32),
        pltpu.VMEM((N + L,), jnp.float32),
        pltpu.VMEM((N + L,), jnp.float32),
        pltpu.VMEM((G * BUFW,), jnp.int32),
        pltpu.VMEM((L,), jnp.float32),
    ],
)
def _density_sc(coords_hbm, out_hbm, xs, ys, zs, xb, yb, zb, sq, ibuf, accv):
    w = lax.axis_index("s") * NC + lax.axis_index("c")
    b = w // W_PER_B
    i0 = (w % W_PER_B) * PTS_PER_W

    pltpu.sync_copy(coords_hbm.at[3 * b + 0], xs.at[pl.ds(0, N)])
    pltpu.sync_copy(coords_hbm.at[3 * b + 1], ys.at[pl.ds(0, N)])
    pltpu.sync_copy(coords_hbm.at[3 * b + 2], zs.at[pl.ds(0, N)])

    def sq_body(c, carry):
        sl = pl.ds(c * L, L)
        xv = xs[sl]
        yv = ys[sl]
        zv = zs[sl]
        sq[sl] = (xv * xv + yv * yv) + zv * zv
        xb[sl] = _rne_bf16(xv)
        yb[sl] = _rne_bf16(yv)
        zb[sl] = _rne_bf16(zv)
        return carry

    lax.fori_loop(0, CHUNKS, sq_body, 0)

    lane = lax.iota(jnp.int32, L)

    def group_body(t, acc):
        ib = i0 + t * G
        ps = []
        for g in range(G):
            sl = pl.ds(ib + g, L)
            ps.append((jnp.float32(2.0) * xb[sl][0],
                       jnp.float32(2.0) * yb[sl][0],
                       jnp.float32(2.0) * zb[sl][0],
                       sq[sl][0]))

        def chunk_body(c, cnts):
            sl = pl.ds(c * L, L)
            xv = xb[sl]
            yv = yb[sl]
            zv = zb[sl]
            sv = sq[sl]
            gidx = c * L + lane
            out = []
            for g in range(G):
                px2, py2, pz2, si = ps[g]
                dot2 = (px2 * xv + py2 * yv) + pz2 * zv
                d = (si + sv) - dot2
                m = d <= R2
                csum = plsc.cumsum(m.astype(jnp.int32))
                plsc.store_scatter(ibuf, [(g * BUFW - 1 + cnts[g]) + csum],
                                   gidx, mask=m)
                out.append(cnts[g] + csum[L - 1])
            return tuple(out)

        cnts = lax.fori_loop(0, CHUNKS, chunk_body,
                             (jnp.int32(0),) * G, unroll=2)

        for g in range(G):
            i = ib + g
            cnt = cnts[g]
            psl = pl.ds(i, L)
            px = xs[psl][0]
            py = ys[psl][0]
            pz = zs[psl][0]
            idx16 = ibuf[pl.ds(g * BUFW, L)]
            kk = jnp.minimum(cnt, NSAMPLE)
            safe = jnp.where(lane < kk, idx16, 0)
            gx = plsc.load_gather(xs, [safe])
            gy = plsc.load_gather(ys, [safe])
            gz = plsc.load_gather(zs, [safe])
            dx = gx - px
            dy = gy - py
            dz = gz - pz
            vals = (dx * dx + dy * dy) + dz * dz
            lsl = pl.ds(N - L, L)
            ex = xs[lsl] - px
            ey = ys[lsl] - py
            ez = zs[lsl] - pz
            dlast = ((ex * ex + ey * ey) + ez * ez)[L - 1]
            first = jnp.where(cnt == 0, dlast, vals[0])
            nine = jnp.where(lane < kk, vals,
                             jnp.where(lane < NSAMPLE, first, INF))
            srt = jnp.sort(nine)
            sel = (lane >= 1) & (lane <= 4)
            v = jnp.where(sel, srt, jnp.float32(1.0))
            v = jnp.maximum(v, jnp.float32(1e-12))
            term = RADIUS - _sqrt16(v) * jnp.exp(-v / H2)
            acc = acc + jnp.where(sel, term, jnp.float32(0.0))
        return acc

    acc = lax.fori_loop(0, GROUPS, group_body, jnp.zeros((L,), jnp.float32))
    accv[...] = acc
    pltpu.sync_copy(accv, out_hbm.at[w])


def kernel(pred):
    coords = jnp.transpose(pred, (0, 2, 1)).reshape(B * 3, N)
    partials = _density_sc(coords)
    return jnp.sum(partials) / np.float32(B * N * 4)


# speedup vs baseline: 28.7055x; 1.0098x over previous
import functools

import jax
import jax.numpy as jnp
import numpy as np
from jax import lax
from jax.experimental import pallas as pl
from jax.experimental.pallas import tpu as pltpu
from jax.experimental.pallas import tpu_sc as plsc

NC = 2
NS = 16
L = 16
NW = NC * NS

B = 4
N = 4096
CHUNKS = N // L
PTS_PER_W = (B * N) // NW
W_PER_B = N // PTS_PER_W
G = 16
GROUPS = PTS_PER_W // G
BUFW = N + L

R2 = np.float32(0.1 ** 2)
H2 = np.float32(0.12 ** 2)
RADIUS = np.float32(0.1)
NSAMPLE = 9
INF = np.float32(np.inf)


def _rne_bf16(x):
    u = plsc.bitcast(x, jnp.int32)
    u = u + jnp.int32(0x7FFF) + (lax.shift_right_logical(u, 16) & jnp.int32(1))
    u = u & jnp.int32(-65536)
    return plsc.bitcast(u, jnp.float32)


def _sqrt16(x):
    i = plsc.bitcast(x, jnp.int32)
    i = jnp.int32(0x1FBD1DF5) + lax.shift_right_arithmetic(i, 1)
    y = plsc.bitcast(i, jnp.float32)
    for _ in range(3):
        y = jnp.float32(0.5) * (y + x / y)
    return y


mesh = plsc.VectorSubcoreMesh(core_axis_name="c", subcore_axis_name="s")


@functools.partial(
    pl.kernel,
    out_type=jax.ShapeDtypeStruct((NW, L), jnp.float32),
    mesh=mesh,
    compiler_params=pltpu.CompilerParams(needs_layout_passes=False),
    scratch_types=[
        pltpu.VMEM((N + L,), jnp.float32),
        pltpu.VMEM((N + L,), jnp.float32),
        pltpu.VMEM((N + L,), jnp.float32),
        pltpu.VMEM((N + L,), jnp.float32),
        pltpu.VMEM((N + L,), jnp.float32),
        pltpu.VMEM((N + L,), jnp.float32),
        pltpu.VMEM((N + L,), jnp.float32),
        pltpu.VMEM((G * BUFW,), jnp.int32),
        pltpu.VMEM((L,), jnp.float32),
    ],
)
def _density_sc(coords_hbm, out_hbm, xs, ys, zs, xb, yb, zb, sq, ibuf, accv):
    w = lax.axis_index("s") * NC + lax.axis_index("c")
    b = w // W_PER_B
    i0 = (w % W_PER_B) * PTS_PER_W

    pltpu.sync_copy(coords_hbm.at[3 * b + 0], xs.at[pl.ds(0, N)])
    pltpu.sync_copy(coords_hbm.at[3 * b + 1], ys.at[pl.ds(0, N)])
    pltpu.sync_copy(coords_hbm.at[3 * b + 2], zs.at[pl.ds(0, N)])

    def sq_body(c, carry):
        sl = pl.ds(c * L, L)
        xv = xs[sl]
        yv = ys[sl]
        zv = zs[sl]
        sq[sl] = (xv * xv + yv * yv) + zv * zv
        xb[sl] = _rne_bf16(xv)
        yb[sl] = _rne_bf16(yv)
        zb[sl] = _rne_bf16(zv)
        return carry

    lax.fori_loop(0, CHUNKS, sq_body, 0)

    lane = lax.iota(jnp.int32, L)

    def group_body(t, acc):
        ib = i0 + t * G
        ps = []
        for g in range(G):
            sl = pl.ds(ib + g, L)
            ps.append((jnp.float32(2.0) * xb[sl][0],
                       jnp.float32(2.0) * yb[sl][0],
                       jnp.float32(2.0) * zb[sl][0],
                       sq[sl][0]))

        def chunk_body(c, cnts):
            sl = pl.ds(c * L, L)
            xv = xb[sl]
            yv = yb[sl]
            zv = zb[sl]
            sv = sq[sl]
            gidx = c * L + lane
            out = []
            for g in range(G):
                px2, py2, pz2, si = ps[g]
                dot2 = (px2 * xv + py2 * yv) + pz2 * zv
                d = (si + sv) - dot2
                m = d <= R2
                csum = plsc.cumsum(m.astype(jnp.int32))
                plsc.store_scatter(ibuf, [(g * BUFW - 1 + cnts[g]) + csum],
                                   gidx, mask=m)
                out.append(cnts[g] + csum[L - 1])
            return tuple(out)

        cnts = lax.fori_loop(0, CHUNKS, chunk_body,
                             (jnp.int32(0),) * G, unroll=1)

        for g in range(G):
            i = ib + g
            cnt = cnts[g]
            psl = pl.ds(i, L)
            px = xs[psl][0]
            py = ys[psl][0]
            pz = zs[psl][0]
            idx16 = ibuf[pl.ds(g * BUFW, L)]
            kk = jnp.minimum(cnt, NSAMPLE)
            safe = jnp.where(lane < kk, idx16, 0)
            gx = plsc.load_gather(xs, [safe])
            gy = plsc.load_gather(ys, [safe])
            gz = plsc.load_gather(zs, [safe])
            dx = gx - px
            dy = gy - py
            dz = gz - pz
            vals = (dx * dx + dy * dy) + dz * dz
            lsl = pl.ds(N - L, L)
            ex = xs[lsl] - px
            ey = ys[lsl] - py
            ez = zs[lsl] - pz
            dlast = ((ex * ex + ey * ey) + ez * ez)[L - 1]
            first = jnp.where(cnt == 0, dlast, vals[0])
            nine = jnp.where(lane < kk, vals,
                             jnp.where(lane < NSAMPLE, first, INF))
            srt = jnp.sort(nine)
            sel = (lane >= 1) & (lane <= 4)
            v = jnp.where(sel, srt, jnp.float32(1.0))
            v = jnp.maximum(v, jnp.float32(1e-12))
            term = RADIUS - _sqrt16(v) * jnp.exp(-v / H2)
            acc = acc + jnp.where(sel, term, jnp.float32(0.0))
        return acc

    acc = lax.fori_loop(0, GROUPS, group_body, jnp.zeros((L,), jnp.float32))
    accv[...] = acc
    pltpu.sync_copy(accv, out_hbm.at[w])


def kernel(pred):
    coords = jnp.transpose(pred, (0, 2, 1)).reshape(B * 3, N)
    partials = _density_sc(coords)
    return jnp.sum(partials) / np.float32(B * N * 4)
